# baseline (speedup 1.0000x reference)
import functools

import jax
import jax.numpy as jnp
from jax import lax
from jax.experimental import pallas as pl
from jax.experimental.pallas import tpu as pltpu
from jax.experimental.pallas import tpu_sc as plsc

_VOCAB = 100000
_EMBED = 64
_BATCH = 1024


def _gather_emb_t(table_t, idx):
    info = plsc.get_sparse_core_info()
    nc, ns = info.num_cores, info.num_subcores
    nw = nc * ns
    b_per_w = _BATCH // nw
    n_chunks = _EMBED * b_per_w // 128
    mesh = plsc.VectorSubcoreMesh(core_axis_name="c", subcore_axis_name="s")

    @functools.partial(
        pl.kernel,
        mesh=mesh,
        out_type=jax.ShapeDtypeStruct((_EMBED, _BATCH), jnp.float32),
        scratch_types=[
            pltpu.VMEM((b_per_w,), jnp.int32),
            pltpu.VMEM((_EMBED, b_per_w), jnp.float32),
            pltpu.SemaphoreType.DMA,
        ],
        compiler_params=pltpu.CompilerParams(use_tc_tiling_on_sc=False),
    )
    def gather_k(tet_hbm, idx_hbm, out_hbm, xv, rows2, sem):
        wid = lax.axis_index("s") * nc + lax.axis_index("c")
        base = wid * b_per_w
        pltpu.sync_copy(idx_hbm.at[pl.ds(base, b_per_w)], xv)
        copies = [
            pltpu.async_copy(tet_hbm.at[d].at[xv], rows2.at[d], sem)
            for d in range(_EMBED)
        ]
        for cp in copies:
            cp.wait()
        pltpu.sync_copy(rows2, out_hbm.at[:, pl.ds(base, b_per_w)])

    return gather_k(table_t, idx)


_VT = 5120
_GRID = (_VOCAB + _VT - 1) // _VT


def _matmul_body(embt_ref, wt_ref, b_ref, out_ref):
    acc = lax.dot_general(
        wt_ref[...],
        embt_ref[...],
        (((0,), (0,)), ((), ())),
        preferred_element_type=jnp.float32,
    )
    out_ref[...] = acc + jnp.transpose(b_ref[...])


def _project_t(emb_t, w_t, bias2d):
    return pl.pallas_call(
        _matmul_body,
        grid=(_GRID,),
        in_specs=[
            pl.BlockSpec((_EMBED, _BATCH), lambda j: (0, 0)),
            pl.BlockSpec((_EMBED, _VT), lambda j: (0, j)),
            pl.BlockSpec((1, _VT), lambda j: (0, j)),
        ],
        out_specs=pl.BlockSpec((_VT, _BATCH), lambda j: (j, 0)),
        out_shape=jax.ShapeDtypeStruct((_VOCAB, _BATCH), jnp.float32),
    )(emb_t, w_t, bias2d)


def kernel(x, token_emb, lm_head_w, lm_head_b):
    emb_t = _gather_emb_t(token_emb.T, x.astype(jnp.int32))
    logits_t = _project_t(emb_t, lm_head_w.T, lm_head_b.reshape(1, _VOCAB))
    return jnp.transpose(logits_t)


# speedup vs baseline: 1.0041x; 1.0041x over previous
import functools

import jax
import jax.numpy as jnp
from jax import lax
from jax.experimental import pallas as pl
from jax.experimental.pallas import tpu as pltpu
from jax.experimental.pallas import tpu_sc as plsc

_VOCAB = 100000
_EMBED = 64
_BATCH = 1024


def _gather_emb_t(table_t, idx):
    info = plsc.get_sparse_core_info()
    nc, ns = info.num_cores, info.num_subcores
    nw = nc * ns
    b_per_w = _BATCH // nw
    n_chunks = _EMBED * b_per_w // 128
    mesh = plsc.VectorSubcoreMesh(core_axis_name="c", subcore_axis_name="s")

    @functools.partial(
        pl.kernel,
        mesh=mesh,
        out_type=jax.ShapeDtypeStruct((_EMBED, _BATCH), jnp.float32),
        scratch_types=[
            pltpu.VMEM((b_per_w,), jnp.int32),
            pltpu.VMEM((_EMBED, b_per_w), jnp.float32),
            pltpu.SemaphoreType.DMA,
        ],
        compiler_params=pltpu.CompilerParams(use_tc_tiling_on_sc=False),
    )
    def gather_k(tet_hbm, idx_hbm, out_hbm, xv, rows2, sem):
        wid = lax.axis_index("s") * nc + lax.axis_index("c")
        base = wid * b_per_w
        pltpu.sync_copy(idx_hbm.at[pl.ds(base, b_per_w)], xv)
        copies = [
            pltpu.async_copy(tet_hbm.at[d].at[xv], rows2.at[d], sem)
            for d in range(_EMBED)
        ]
        for cp in copies:
            cp.wait()
        pltpu.sync_copy(rows2, out_hbm.at[:, pl.ds(base, b_per_w)])

    return gather_k(table_t, idx)


_VT = 4096
_GRID = (_VOCAB + _VT - 1) // _VT


def _matmul_body(embt_ref, wt_ref, b_ref, out_ref):
    acc = lax.dot_general(
        wt_ref[...],
        embt_ref[...],
        (((0,), (0,)), ((), ())),
        preferred_element_type=jnp.float32,
    )
    out_ref[...] = acc + jnp.transpose(b_ref[...])


def _project_t(emb_t, w_t, bias2d):
    return pl.pallas_call(
        _matmul_body,
        grid=(_GRID,),
        in_specs=[
            pl.BlockSpec((_EMBED, _BATCH), lambda j: (0, 0)),
            pl.BlockSpec((_EMBED, _VT), lambda j: (0, j)),
            pl.BlockSpec((1, _VT), lambda j: (0, j)),
        ],
        out_specs=pl.BlockSpec((_VT, _BATCH), lambda j: (j, 0)),
        out_shape=jax.ShapeDtypeStruct((_VOCAB, _BATCH), jnp.float32),
    )(emb_t, w_t, bias2d)


def kernel(x, token_emb, lm_head_w, lm_head_b):
    emb_t = _gather_emb_t(token_emb.T, x.astype(jnp.int32))
    logits_t = _project_t(emb_t, lm_head_w.T, lm_head_b.reshape(1, _VOCAB))
    return jnp.transpose(logits_t)
